# baseline (speedup 1.0000x reference)
import functools

import jax
import jax.numpy as jnp
from jax import lax
from jax.experimental import pallas as pl
from jax.experimental.pallas import tpu as pltpu
from jax.experimental.pallas import tpu_sc as plsc

N = 10000
S = 32
H = 128
NPAD = 10240
NC, NS = 2, 16
NW = NC * NS
NB = NPAD // NW
EPW = NB * S

@functools.cache
def _mesh():
    return plsc.VectorSubcoreMesh(
        core_axis_name="c", subcore_axis_name="s",
        num_cores=NC, num_subcores=NS)


def _wid():
    return lax.axis_index("s") * NC + lax.axis_index("c")


def _sc_gather(table, idx, ch, nchunk):
    v, d = table.shape
    bw = ch * nchunk
    b = idx.shape[0]
    assert b == bw * NW and ch % 8 == 0

    @functools.partial(
        pl.kernel,
        out_type=jax.ShapeDtypeStruct((b, d), jnp.float32),
        mesh=_mesh(),
        compiler_params=pltpu.CompilerParams(needs_layout_passes=False),
        scratch_types=[
            pltpu.VMEM((bw,), jnp.int32),
            pltpu.VMEM((2, ch, d), jnp.float32),
            pltpu.SemaphoreType.DMA,
            pltpu.SemaphoreType.DMA,
        ],
    )
    def k(table_hbm, idx_hbm, out_hbm, idx_v, rows_v, sem0, sem1):
        base = _wid() * bw
        pltpu.sync_copy(idx_hbm.at[pl.ds(base, bw)], idx_v)
        sems = (sem0, sem1)

        def fire(c, buf):
            pltpu.make_async_copy(
                table_hbm.at[idx_v.at[pl.ds(c * ch, ch)]],
                rows_v.at[buf], sems[buf]).start()

        fire(0, 0)
        for c in range(nchunk):
            buf = c % 2
            if c + 1 < nchunk:
                fire(c + 1, 1 - buf)
            pltpu.make_async_copy(
                table_hbm.at[idx_v.at[pl.ds(0, ch)]],
                rows_v.at[buf], sems[buf]).wait()
            pltpu.sync_copy(rows_v.at[buf],
                            out_hbm.at[pl.ds(base + c * ch, ch)])

    return k(table, idx)


def _sc_pool(z, e_vec, nbt_flat, nsplit):
    d = z.shape[1]
    nbq = NB // nsplit
    assert nbq <= 128 and nbq % 16 == 0
    epq = nbq * S
    dv = d // 16
    ngr = nbq // 16
    nacc = 8

    @functools.partial(
        pl.kernel,
        out_type=(jax.ShapeDtypeStruct((NPAD, d), jnp.float32),
                  jax.ShapeDtypeStruct((NPAD,), jnp.float32)),
        mesh=_mesh(),
        compiler_params=pltpu.CompilerParams(needs_layout_passes=False),
        scratch_types=[
            pltpu.VMEM((S, nbq), jnp.int32),
            pltpu.VMEM((NPAD,), jnp.float32),
            pltpu.VMEM((nbq,), jnp.float32),
            pltpu.VMEM((nacc, nbq, d), jnp.float32),
        ] + [pltpu.SemaphoreType.DMA] * nacc,
    )
    def k(z_hbm, e_hbm, nbt_hbm, out_hbm, den_hbm,
          idx_v, e_v, den_v, acc_v, *gsem):
        w = _wid()
        pltpu.sync_copy(e_hbm, e_v)
        zv = jnp.zeros((16,), jnp.float32)

        def fire(sg, par):
            pltpu.async_copy(
                z_hbm.at[idx_v.at[sg]],
                acc_v.at[par], gsem[par], add=True)

        def wait_g(par):
            pltpu.make_async_copy(
                z_hbm.at[idx_v.at[0]],
                acc_v.at[par], gsem[par]).wait()

        for q in range(nsplit):
            pltpu.sync_copy(nbt_hbm.at[pl.ds((q * NW + w) * S, S)], idx_v)

            def zbody(r, carry):
                for par in range(nacc):
                    for dd in range(dv):
                        acc_v[par, r, pl.ds(dd * 16, 16)] = zv
                return carry
            lax.fori_loop(0, nbq, zbody, 0)

            for par in range(nacc):
                fire(par, par)

            def den_body(g, carry):
                acc = jnp.zeros((16,), jnp.float32)
                for sg in range(S):
                    ii = idx_v[sg, pl.ds(g * 16, 16)]
                    acc = acc + plsc.load_gather(e_v, [ii])
                den_v[pl.ds(g * 16, 16)] = acc
                return carry
            lax.fori_loop(0, ngr, den_body, 0)

            def sbody(ss, carry):
                for par in range(nacc):
                    sg = nacc + ss * nacc + par
                    wait_g(par)
                    fire(sg, par)
                return carry
            lax.fori_loop(0, (S - nacc) // nacc, sbody, 0)
            for par in range(nacc):
                wait_g(par)

            def mbody(r, carry):
                for dd in range(dv):
                    v = acc_v[0, r, pl.ds(dd * 16, 16)]
                    for par in range(1, nacc):
                        v = v + acc_v[par, r, pl.ds(dd * 16, 16)]
                    acc_v[0, r, pl.ds(dd * 16, 16)] = v
                return carry
            lax.fori_loop(0, nbq, mbody, 0)
            obase = q * NW * nbq + w * nbq
            pltpu.sync_copy(acc_v.at[0], out_hbm.at[pl.ds(obase, nbq)])
            pltpu.sync_copy(den_v, den_hbm.at[pl.ds(obase, nbq)])

    return k(z, e_vec, nbt_flat.reshape(-1, nbq))


_RA = 512


def _lstm_body(emb_ref, wi_ref, wh_ref, b_ref, a2_ref, h_ref, p2_ref):
    pid = pl.program_id(0)
    x = emb_ref[...]
    wi = wi_ref[...]
    wh = wh_ref[...]
    bb = b_ref[...]
    h = jnp.zeros((_RA, H), jnp.float32)
    c = jnp.zeros((_RA, H), jnp.float32)
    for t in range(4):
        xt = x[:, t * H:(t + 1) * H]
        g = (jnp.dot(xt, wi, preferred_element_type=jnp.float32)
             + jnp.dot(h, wh, preferred_element_type=jnp.float32) + bb)
        gi = g[:, :H]
        gf = g[:, H:2 * H]
        gg = g[:, 2 * H:3 * H]
        go = g[:, 3 * H:]
        c = jax.nn.sigmoid(gf) * c + jax.nn.sigmoid(gi) * jnp.tanh(gg)
        h = jax.nn.sigmoid(go) * jnp.tanh(c)
    rows = pid * _RA + lax.broadcasted_iota(jnp.int32, (_RA, 1), 0)
    h = jnp.where(rows < N, h, 0.0)
    h_ref[...] = h
    p2_ref[...] = jnp.dot(h, a2_ref[...], preferred_element_type=jnp.float32)


def _tc_lstm(emb2, wi, wh, b, a2):
    return pl.pallas_call(
        _lstm_body,
        grid=(NPAD // _RA,),
        in_specs=[
            pl.BlockSpec((_RA, 4 * H), lambda i: (i, 0)),
            pl.BlockSpec((H, 4 * H), lambda i: (0, 0)),
            pl.BlockSpec((H, 4 * H), lambda i: (0, 0)),
            pl.BlockSpec((1, 4 * H), lambda i: (0, 0)),
            pl.BlockSpec((H, 2), lambda i: (0, 0)),
        ],
        out_specs=[
            pl.BlockSpec((_RA, H), lambda i: (i, 0)),
            pl.BlockSpec((_RA, 2), lambda i: (i, 0)),
        ],
        out_shape=[
            jax.ShapeDtypeStruct((NPAD, H), jnp.float32),
            jax.ShapeDtypeStruct((NPAD, 2), jnp.float32),
        ],
    )(emb2, wi, wh, b, a2)


def _seq_body(x_ref, wif_ref, bf_ref, wib_ref, bb_ref, out_ref):
    x = x_ref[...]

    def cell(wi, b):
        g = jnp.dot(x, wi, preferred_element_type=jnp.float32) + b
        gi = g[:, :H]
        gg = g[:, 2 * H:3 * H]
        go = g[:, 3 * H:]
        c = jax.nn.sigmoid(gi) * jnp.tanh(gg)
        return jax.nn.sigmoid(go) * jnp.tanh(c)

    out_ref[...] = jnp.concatenate(
        [cell(wif_ref[...], bf_ref[...]), cell(wib_ref[...], bb_ref[...])],
        axis=1)


def _tc_seq(x0, wif, bf, wib, bb):
    return pl.pallas_call(
        _seq_body,
        out_shape=jax.ShapeDtypeStruct((16, 2 * H), jnp.float32),
    )(x0, wif, bf, wib, bb)


def _make_scale_body(nh):
    def body(*refs):
        xa_ref, pa_ref, xb_ref, pb_ref = refs[:4]
        outs = refs[4:]
        pid = pl.program_id(0)

        def one(x_ref, p_ref, z_refs, e_ref):
            m = jnp.max(p_ref[...])
            e = jnp.exp(p_ref[pl.ds(pid * _RB, _RB), :] - m)
            for hh, z_ref in enumerate(z_refs):
                z_ref[...] = e * x_ref[:, hh * H:(hh + 1) * H]
            e_ref[...] = e

        one(xa_ref, pa_ref, outs[:nh], outs[nh])
        one(xb_ref, pb_ref, outs[nh + 1:2 * nh + 1], outs[2 * nh + 1])
    return body


def _tc_scale2(xa, pa, xb, pb):
    d = xa.shape[1]
    nh = d // H
    zspec = [pl.BlockSpec((_RB, H), lambda i: (i, 0)) for _ in range(nh)]
    zshape = [jax.ShapeDtypeStruct((NPAD, H), jnp.float32)
              for _ in range(nh)]
    espec = [pl.BlockSpec((_RB, 1), lambda i: (i, 0))]
    eshape = [jax.ShapeDtypeStruct((NPAD, 1), jnp.float32)]
    return pl.pallas_call(
        _make_scale_body(nh),
        grid=(NPAD // _RB,),
        in_specs=[
            pl.BlockSpec((_RB, d), lambda i: (i, 0)),
            pl.BlockSpec((NPAD, 1), lambda i: (0, 0)),
            pl.BlockSpec((_RB, d), lambda i: (i, 0)),
            pl.BlockSpec((NPAD, 1), lambda i: (0, 0)),
        ],
        out_specs=zspec + espec + zspec + espec,
        out_shape=zshape + eshape + zshape + eshape,
    )(xa, pa, xb, pb)


def _gated_out(x, pooled, ws, wn, wg, ug, b):
    gate = jax.nn.sigmoid(
        jnp.dot(x, wg, preferred_element_type=jnp.float32)
        + jnp.dot(pooled, ug, preferred_element_type=jnp.float32))
    gp = gate * pooled
    h = jnp.concatenate(
        [jnp.dot(x, ws, preferred_element_type=jnp.float32),
         jnp.dot(gp, wn, preferred_element_type=jnp.float32)], axis=1) + b
    return jnp.maximum(h, 0.0)


_RB = 512


def _agg0_body(x_ref, pf_ref, df_ref, pb_ref, db_ref,
               fws, fwn, fwg, fug, fb,
               bws, bwn, bwg, bug, bb_,
               a1f_ref, a1b_ref,
               x1f_ref, x1b_ref, p1f_ref, p1b_ref):
    pid = pl.program_id(0)
    x = x_ref[...]
    rows = pid * _RB + lax.broadcasted_iota(jnp.int32, (_RB, 1), 0)
    mask = rows < N
    pf = pf_ref[...] / df_ref[...]
    pb = pb_ref[...] / db_ref[...]
    hf = _gated_out(x, pf, fws[...], fwn[...], fwg[...], fug[...],
                    fb[...])
    hb = _gated_out(x, pb, bws[...], bwn[...], bwg[...], bug[...],
                    bb_[...])
    hf = jnp.where(mask, hf, 0.0)
    hb = jnp.where(mask, hb, 0.0)
    x1f_ref[...] = hf
    x1b_ref[...] = hb
    p1f_ref[...] = jnp.dot(hf, a1f_ref[...], preferred_element_type=jnp.float32)
    p1b_ref[...] = jnp.dot(hb, a1b_ref[...], preferred_element_type=jnp.float32)


def _tc_agg0(x, pf, df, pb, db, fw_w, bw_w, a1f, a1b):
    wspec = [
        pl.BlockSpec((H, H), lambda i: (0, 0)),
        pl.BlockSpec((H, H), lambda i: (0, 0)),
        pl.BlockSpec((H, H), lambda i: (0, 0)),
        pl.BlockSpec((H, H), lambda i: (0, 0)),
        pl.BlockSpec((1, 2 * H), lambda i: (0, 0)),
    ]
    return pl.pallas_call(
        _agg0_body,
        grid=(NPAD // _RB,),
        in_specs=[
            pl.BlockSpec((_RB, H), lambda i: (i, 0)),
            pl.BlockSpec((_RB, H), lambda i: (i, 0)),
            pl.BlockSpec((_RB, 1), lambda i: (i, 0)),
            pl.BlockSpec((_RB, H), lambda i: (i, 0)),
            pl.BlockSpec((_RB, 1), lambda i: (i, 0)),
        ] + wspec + wspec + [
            pl.BlockSpec((2 * H, 1), lambda i: (0, 0)),
            pl.BlockSpec((2 * H, 1), lambda i: (0, 0)),
        ],
        out_specs=[
            pl.BlockSpec((_RB, 2 * H), lambda i: (i, 0)),
            pl.BlockSpec((_RB, 2 * H), lambda i: (i, 0)),
            pl.BlockSpec((_RB, 1), lambda i: (i, 0)),
            pl.BlockSpec((_RB, 1), lambda i: (i, 0)),
        ],
        out_shape=[
            jax.ShapeDtypeStruct((NPAD, 2 * H), jnp.float32),
            jax.ShapeDtypeStruct((NPAD, 2 * H), jnp.float32),
            jax.ShapeDtypeStruct((NPAD, 1), jnp.float32),
            jax.ShapeDtypeStruct((NPAD, 1), jnp.float32),
        ],
    )(x, pf, df, pb, db, *fw_w, *bw_w, a1f, a1b)


_PG = 200
_NG = 50


def _agg1_body(xf_ref, xb_ref, pf_ref, df_ref, pb_ref, db_ref,
               fws, fwn, fwg, fug, fb,
               bws, bwn, bwg, bug, bb_,
               gh_ref, ge_ref):
    pf = pf_ref[...] / df_ref[...]
    pb = pb_ref[...] / db_ref[...]
    hf = _gated_out(xf_ref[...], pf, fws[...], fwn[...], fwg[...],
                    fug[...], fb[...])
    hb = _gated_out(xb_ref[...], pb, bws[...], bwn[...], bwg[...],
                    bug[...], bb_[...])
    hid = jnp.concatenate([hf, hb], axis=1)
    gh_ref[...] = hid[None]
    ge_ref[...] = jnp.max(hid, axis=0, keepdims=True)[None]


def _tc_agg1(xf, xb, pf, df, pb, db, fw_w, bw_w):
    wspec = [
        pl.BlockSpec((2 * H, H), lambda i: (0, 0)),
        pl.BlockSpec((2 * H, H), lambda i: (0, 0)),
        pl.BlockSpec((2 * H, 2 * H), lambda i: (0, 0)),
        pl.BlockSpec((2 * H, 2 * H), lambda i: (0, 0)),
        pl.BlockSpec((1, 2 * H), lambda i: (0, 0)),
    ]
    return pl.pallas_call(
        _agg1_body,
        grid=(_NG,),
        in_specs=[
            pl.BlockSpec((_PG, 2 * H), lambda i: (i, 0)),
            pl.BlockSpec((_PG, 2 * H), lambda i: (i, 0)),
            pl.BlockSpec((_PG, 2 * H), lambda i: (i, 0)),
            pl.BlockSpec((_PG, 1), lambda i: (i, 0)),
            pl.BlockSpec((_PG, 2 * H), lambda i: (i, 0)),
            pl.BlockSpec((_PG, 1), lambda i: (i, 0)),
        ] + wspec + wspec,
        out_specs=[
            pl.BlockSpec((1, _PG, 4 * H), lambda i: (i, 0, 0)),
            pl.BlockSpec((1, 1, 4 * H), lambda i: (i, 0, 0)),
        ],
        out_shape=[
            jax.ShapeDtypeStruct((_NG, _PG, 4 * H), jnp.float32),
            jax.ShapeDtypeStruct((_NG, 1, 4 * H), jnp.float32),
        ],
    )(xf, xb, pf, df, pb, db, *fw_w, *bw_w)


def kernel(batch_nodes, node_features, fw_adj, bw_adj, idx_sql_seqs,
           sql_seqs_lens, embed_table,
           Wi_sf, Wh_sf, b_sf, Wi_sb, Wh_sb, b_sb, Wi_nn, Wh_nn, b_nn,
           fw0_Ws, fw0_Wn, fw0_a, fw0_Wg, fw0_Ug, fw0_b,
           bw0_Ws, bw0_Wn, bw0_a, bw0_Wg, bw0_Ug, bw0_b,
           fw1_Ws, fw1_Wn, fw1_a, fw1_Wg, fw1_Ug, fw1_b,
           bw1_Ws, bw1_Wn, bw1_a, bw1_Wg, bw1_Ug, bw1_b):
    f32 = jnp.float32

    nf_pad = jnp.pad(node_features, ((0, NPAD - N), (0, 0)))
    idx_all = jnp.concatenate([
        nf_pad.reshape(-1),
        idx_sql_seqs[:, 0],
        jnp.zeros((41216 - NPAD * 4 - 16,), jnp.int32),
    ])
    def nbt(adj, nsplit):
        nbq = NB // nsplit
        return (jnp.pad(adj[:-1], ((0, NPAD - N), (0, 0)))
                .reshape(nsplit, NW, nbq, S).transpose(0, 1, 3, 2)
                .reshape(-1))
    nbf4, nbb4 = nbt(fw_adj, 4), nbt(bw_adj, 4)

    rows = _sc_gather(embed_table, idx_all, ch=184, nchunk=7)
    emb2 = rows[:NPAD * 4].reshape(NPAD, 4 * H)
    x0 = rows[NPAD * 4:NPAD * 4 + 16]

    a2 = jnp.stack([fw0_a, bw0_a], axis=1)
    h0, p02 = _tc_lstm(emb2, Wi_nn, Wh_nn, b_nn[None], a2)
    seqs = _tc_seq(x0, Wi_sf, b_sf[None], Wi_sb, b_sb[None])

    z0f, e0f, z0b, e0b = _tc_scale2(h0, p02[:, :1], h0, p02[:, 1:])
    pool0f, den0f = _sc_pool(z0f, e0f.reshape(-1), nbf4, nsplit=4)
    pool0b, den0b = _sc_pool(z0b, e0b.reshape(-1), nbb4, nsplit=4)
    x1f, x1b, p1f2, p1b2 = _tc_agg0(
        h0, pool0f, den0f[:, None], pool0b, den0b[:, None],
        (fw0_Ws, fw0_Wn, fw0_Wg, fw0_Ug, fw0_b[None]),
        (bw0_Ws, bw0_Wn, bw0_Wg, bw0_Ug, bw0_b[None]),
        fw1_a[:, None], bw1_a[:, None])

    z1f_lo, z1f_hi, e1f, z1b_lo, z1b_hi, e1b = _tc_scale2(
        x1f, p1f2, x1b, p1b2)
    pool1f_lo, den1f = _sc_pool(z1f_lo, e1f.reshape(-1), nbf4, nsplit=4)
    pool1f_hi, _ = _sc_pool(z1f_hi, e1f.reshape(-1), nbf4, nsplit=4)
    pool1b_lo, den1b = _sc_pool(z1b_lo, e1b.reshape(-1), nbb4, nsplit=4)
    pool1b_hi, _ = _sc_pool(z1b_hi, e1b.reshape(-1), nbb4, nsplit=4)
    pool1f = jnp.concatenate([pool1f_lo, pool1f_hi], axis=1)
    pool1b = jnp.concatenate([pool1b_lo, pool1b_hi], axis=1)
    graph_hidden, graph_embedding3 = _tc_agg1(
        x1f, x1b, pool1f, den1f[:, None], pool1b, den1b[:, None],
        (fw1_Ws, fw1_Wn, fw1_Wg, fw1_Ug, fw1_b[None]),
        (bw1_Ws, bw1_Wn, bw1_Wg, bw1_Ug, bw1_b[None]))
    graph_embedding = graph_embedding3[:, 0, :]

    seqs_encoding = seqs[:, None, :]
    seqs_encoding_mask = (idx_sql_seqs == 0)
    return (graph_hidden, graph_embedding, _PG, seqs_encoding,
            seqs_encoding_mask)


# speedup vs baseline: 1.2437x; 1.2437x over previous
import functools

import jax
import jax.numpy as jnp
from jax import lax
from jax.experimental import pallas as pl
from jax.experimental.pallas import tpu as pltpu
from jax.experimental.pallas import tpu_sc as plsc

N = 10000
S = 32
H = 128
NPAD = 10240
NC, NS = 2, 16
NW = NC * NS
NB = NPAD // NW
EPW = NB * S


@functools.cache
def _mesh():
    return plsc.VectorSubcoreMesh(
        core_axis_name="c", subcore_axis_name="s",
        num_cores=NC, num_subcores=NS)


def _wid():
    return lax.axis_index("s") * NC + lax.axis_index("c")


def _sc_gather(table, idx, ch, nchunk):
    v, d = table.shape
    bw = ch * nchunk
    b = idx.shape[0]
    assert b == bw * NW and ch % 8 == 0

    @functools.partial(
        pl.kernel,
        out_type=jax.ShapeDtypeStruct((b, d), jnp.float32),
        mesh=_mesh(),
        compiler_params=pltpu.CompilerParams(needs_layout_passes=False),
        scratch_types=[
            pltpu.VMEM((bw,), jnp.int32),
            pltpu.VMEM((2, ch, d), jnp.float32),
            pltpu.SemaphoreType.DMA,
            pltpu.SemaphoreType.DMA,
        ],
    )
    def k(table_hbm, idx_hbm, out_hbm, idx_v, rows_v, sem0, sem1):
        base = _wid() * bw
        pltpu.sync_copy(idx_hbm.at[pl.ds(base, bw)], idx_v)
        sems = (sem0, sem1)

        def fire(c, buf):
            pltpu.make_async_copy(
                table_hbm.at[idx_v.at[pl.ds(c * ch, ch)]],
                rows_v.at[buf], sems[buf]).start()

        fire(0, 0)
        for c in range(nchunk):
            buf = c % 2
            if c + 1 < nchunk:
                fire(c + 1, 1 - buf)
            pltpu.make_async_copy(
                table_hbm.at[idx_v.at[pl.ds(0, ch)]],
                rows_v.at[buf], sems[buf]).wait()
            pltpu.sync_copy(rows_v.at[buf],
                            out_hbm.at[pl.ds(base + c * ch, ch)])

    return k(table, idx)


def _sc_pool(x, nb_flat, p, c_nodes):
    d = x.shape[1]
    dv = d // 16
    nch = NB // c_nodes
    assert nch % 2 == 0
    cs = c_nodes * S

    @functools.partial(
        pl.kernel,
        out_type=jax.ShapeDtypeStruct((NPAD, d), jnp.float32),
        mesh=_mesh(),
        compiler_params=pltpu.CompilerParams(needs_layout_passes=False),
        scratch_types=[
            pltpu.VMEM((EPW,), jnp.int32),
            pltpu.VMEM((NPAD,), jnp.float32),
            pltpu.VMEM((EPW,), jnp.float32),
            pltpu.VMEM((2, cs, d), jnp.float32),
            pltpu.VMEM((2, c_nodes, d), jnp.float32),
            pltpu.SemaphoreType.DMA,
            pltpu.SemaphoreType.DMA,
            pltpu.SemaphoreType.DMA,
            pltpu.SemaphoreType.DMA,
        ],
    )
    def k(x_hbm, nb_hbm, p_hbm, out_hbm, idx_v, p_v, a_v, rows_v, st_v,
          g0, g1, o0, o1):
        w = _wid()
        nbase = w * NB
        pltpu.sync_copy(nb_hbm.at[pl.ds(w * EPW, EPW)], idx_v)
        pltpu.sync_copy(p_hbm, p_v)
        gsem = (g0, g1)
        osem = (o0, o1)

        def sm_body(i, carry):
            off = pl.multiple_of(i * S, 8)
            i1 = idx_v[pl.ds(off, 16)]
            i2 = idx_v[pl.ds(off + 16, 16)]
            s1 = plsc.load_gather(p_v, [i1])
            s2 = plsc.load_gather(p_v, [i2])
            m = jnp.max(jnp.maximum(s1, s2))
            e1 = jnp.exp(s1 - m)
            e2 = jnp.exp(s2 - m)
            dvec = lax.broadcast_in_dim(jnp.sum(e1 + e2), (16,), ())
            a_v[pl.ds(off, 16)] = e1 / dvec
            a_v[pl.ds(off + 16, 16)] = e2 / dvec
            return carry
        lax.fori_loop(0, NB, sm_body, 0)

        def fire(c, buf):
            off = pl.multiple_of(c * cs, 8)
            pltpu.make_async_copy(
                x_hbm.at[idx_v.at[pl.ds(off, cs)]],
                rows_v.at[buf], gsem[buf]).start()

        def wait_gather(buf):
            pltpu.make_async_copy(
                x_hbm.at[idx_v.at[pl.ds(0, cs)]],
                rows_v.at[buf], gsem[buf]).wait()

        def wait_out(buf):
            pltpu.make_async_copy(
                st_v.at[buf],
                out_hbm.at[pl.ds(nbase, c_nodes)], osem[buf]).wait()

        fire(0, 0)

        def chunk_body(cc, carry):
            for bparity in range(2):
                c = cc * 2 + bparity

                @pl.when(c + 1 < nch)
                def _():
                    fire(c + 1, 1 - bparity)

                wait_gather(bparity)

                @pl.when(c >= 2)
                def _():
                    wait_out(bparity)

                for j in range(c_nodes):
                    abase = pl.multiple_of((c * c_nodes + j) * S, 8)

                    def s_body(s4, accs):
                        accs = list(accs)
                        for u in range(4):
                            srel = s4 * 4 + u
                            aw = plsc.load_gather(
                                a_v,
                                [jnp.full((16,), abase + srel, jnp.int32)])
                            for dd in range(dv):
                                r = rows_v[bparity, j * S + srel,
                                           pl.ds(dd * 16, 16)]
                                accs[dd] = accs[dd] + aw * r
                        return tuple(accs)

                    accs0 = tuple(jnp.zeros((16,), jnp.float32)
                                  for _ in range(dv))
                    accs = lax.fori_loop(0, S // 4, s_body, accs0)
                    for dd in range(dv):
                        st_v[bparity, j, pl.ds(dd * 16, 16)] = accs[dd]

                pltpu.make_async_copy(
                    st_v.at[bparity],
                    out_hbm.at[pl.ds(nbase + c * c_nodes, c_nodes)],
                    osem[bparity]).start()
            return carry

        lax.fori_loop(0, nch // 2, chunk_body, 0)
        wait_out(0)
        wait_out(1)

    return k(x, nb_flat, p)


_RA = 512


def _lstm_body(emb_ref, wi_ref, wh_ref, b_ref, a2_ref, h_ref, p2_ref):
    pid = pl.program_id(0)
    x = emb_ref[...]
    wi = wi_ref[...]
    wh = wh_ref[...]
    bb = b_ref[...]
    h = jnp.zeros((_RA, H), jnp.float32)
    c = jnp.zeros((_RA, H), jnp.float32)
    for t in range(4):
        xt = x[:, t * H:(t + 1) * H]
        g = (jnp.dot(xt, wi, preferred_element_type=jnp.float32)
             + jnp.dot(h, wh, preferred_element_type=jnp.float32) + bb)
        gi = g[:, :H]
        gf = g[:, H:2 * H]
        gg = g[:, 2 * H:3 * H]
        go = g[:, 3 * H:]
        c = jax.nn.sigmoid(gf) * c + jax.nn.sigmoid(gi) * jnp.tanh(gg)
        h = jax.nn.sigmoid(go) * jnp.tanh(c)
    rows = pid * _RA + lax.broadcasted_iota(jnp.int32, (_RA, 1), 0)
    h = jnp.where(rows < N, h, 0.0)
    h_ref[...] = h
    p2_ref[...] = jnp.dot(h, a2_ref[...], preferred_element_type=jnp.float32)


def _tc_lstm(emb2, wi, wh, b, a2):
    return pl.pallas_call(
        _lstm_body,
        grid=(NPAD // _RA,),
        in_specs=[
            pl.BlockSpec((_RA, 4 * H), lambda i: (i, 0)),
            pl.BlockSpec((H, 4 * H), lambda i: (0, 0)),
            pl.BlockSpec((H, 4 * H), lambda i: (0, 0)),
            pl.BlockSpec((1, 4 * H), lambda i: (0, 0)),
            pl.BlockSpec((H, 2), lambda i: (0, 0)),
        ],
        out_specs=[
            pl.BlockSpec((_RA, H), lambda i: (i, 0)),
            pl.BlockSpec((_RA, 2), lambda i: (i, 0)),
        ],
        out_shape=[
            jax.ShapeDtypeStruct((NPAD, H), jnp.float32),
            jax.ShapeDtypeStruct((NPAD, 2), jnp.float32),
        ],
    )(emb2, wi, wh, b, a2)


def _seq_body(x_ref, wif_ref, bf_ref, wib_ref, bb_ref, out_ref):
    x = x_ref[...]

    def cell(wi, b):
        g = jnp.dot(x, wi, preferred_element_type=jnp.float32) + b
        gi = g[:, :H]
        gg = g[:, 2 * H:3 * H]
        go = g[:, 3 * H:]
        c = jax.nn.sigmoid(gi) * jnp.tanh(gg)
        return jax.nn.sigmoid(go) * jnp.tanh(c)

    out_ref[...] = jnp.concatenate(
        [cell(wif_ref[...], bf_ref[...]), cell(wib_ref[...], bb_ref[...])],
        axis=1)


def _tc_seq(x0, wif, bf, wib, bb):
    return pl.pallas_call(
        _seq_body,
        out_shape=jax.ShapeDtypeStruct((16, 2 * H), jnp.float32),
    )(x0, wif, bf, wib, bb)


def _gated_out(x, pooled, ws, wn, wg, ug, b):
    gate = jax.nn.sigmoid(
        jnp.dot(x, wg, preferred_element_type=jnp.float32)
        + jnp.dot(pooled, ug, preferred_element_type=jnp.float32))
    gp = gate * pooled
    h = jnp.concatenate(
        [jnp.dot(x, ws, preferred_element_type=jnp.float32),
         jnp.dot(gp, wn, preferred_element_type=jnp.float32)], axis=1) + b
    return jnp.maximum(h, 0.0)


_RB = 512


def _agg0_body(x_ref, pf_ref, pb_ref,
               fws, fwn, fwg, fug, fb,
               bws, bwn, bwg, bug, bb_,
               a1f_ref, a1b_ref,
               x1f_ref, x1b_ref, p1f_ref, p1b_ref):
    pid = pl.program_id(0)
    x = x_ref[...]
    rows = pid * _RB + lax.broadcasted_iota(jnp.int32, (_RB, 1), 0)
    mask = rows < N
    hf = _gated_out(x, pf_ref[...], fws[...], fwn[...], fwg[...], fug[...],
                    fb[...])
    hb = _gated_out(x, pb_ref[...], bws[...], bwn[...], bwg[...], bug[...],
                    bb_[...])
    hf = jnp.where(mask, hf, 0.0)
    hb = jnp.where(mask, hb, 0.0)
    x1f_ref[...] = hf
    x1b_ref[...] = hb
    p1f_ref[...] = jnp.dot(hf, a1f_ref[...],
                           preferred_element_type=jnp.float32)
    p1b_ref[...] = jnp.dot(hb, a1b_ref[...],
                           preferred_element_type=jnp.float32)


def _tc_agg0(x, pf, pb, fw_w, bw_w, a1f, a1b):
    wspec = [
        pl.BlockSpec((H, H), lambda i: (0, 0)),
        pl.BlockSpec((H, H), lambda i: (0, 0)),
        pl.BlockSpec((H, H), lambda i: (0, 0)),
        pl.BlockSpec((H, H), lambda i: (0, 0)),
        pl.BlockSpec((1, 2 * H), lambda i: (0, 0)),
    ]
    return pl.pallas_call(
        _agg0_body,
        grid=(NPAD // _RB,),
        in_specs=[
            pl.BlockSpec((_RB, H), lambda i: (i, 0)),
            pl.BlockSpec((_RB, H), lambda i: (i, 0)),
            pl.BlockSpec((_RB, H), lambda i: (i, 0)),
        ] + wspec + wspec + [
            pl.BlockSpec((2 * H, 1), lambda i: (0, 0)),
            pl.BlockSpec((2 * H, 1), lambda i: (0, 0)),
        ],
        out_specs=[
            pl.BlockSpec((_RB, 2 * H), lambda i: (i, 0)),
            pl.BlockSpec((_RB, 2 * H), lambda i: (i, 0)),
            pl.BlockSpec((_RB, 1), lambda i: (i, 0)),
            pl.BlockSpec((_RB, 1), lambda i: (i, 0)),
        ],
        out_shape=[
            jax.ShapeDtypeStruct((NPAD, 2 * H), jnp.float32),
            jax.ShapeDtypeStruct((NPAD, 2 * H), jnp.float32),
            jax.ShapeDtypeStruct((NPAD, 1), jnp.float32),
            jax.ShapeDtypeStruct((NPAD, 1), jnp.float32),
        ],
    )(x, pf, pb, *fw_w, *bw_w, a1f, a1b)


_PG = 200
_NG = 50


def _agg1_body(xf_ref, xb_ref, pf_ref, pb_ref,
               fws, fwn, fwg, fug, fb,
               bws, bwn, bwg, bug, bb_,
               gh_ref, ge_ref):
    hf = _gated_out(xf_ref[...], pf_ref[...], fws[...], fwn[...], fwg[...],
                    fug[...], fb[...])
    hb = _gated_out(xb_ref[...], pb_ref[...], bws[...], bwn[...], bwg[...],
                    bug[...], bb_[...])
    hid = jnp.concatenate([hf, hb], axis=1)
    gh_ref[...] = hid[None]
    ge_ref[...] = jnp.max(hid, axis=0, keepdims=True)[None]


def _tc_agg1(xf, xb, pf, pb, fw_w, bw_w):
    wspec = [
        pl.BlockSpec((2 * H, H), lambda i: (0, 0)),
        pl.BlockSpec((2 * H, H), lambda i: (0, 0)),
        pl.BlockSpec((2 * H, 2 * H), lambda i: (0, 0)),
        pl.BlockSpec((2 * H, 2 * H), lambda i: (0, 0)),
        pl.BlockSpec((1, 2 * H), lambda i: (0, 0)),
    ]
    return pl.pallas_call(
        _agg1_body,
        grid=(_NG,),
        in_specs=[
            pl.BlockSpec((_PG, 2 * H), lambda i: (i, 0)),
            pl.BlockSpec((_PG, 2 * H), lambda i: (i, 0)),
            pl.BlockSpec((_PG, 2 * H), lambda i: (i, 0)),
            pl.BlockSpec((_PG, 2 * H), lambda i: (i, 0)),
        ] + wspec + wspec,
        out_specs=[
            pl.BlockSpec((1, _PG, 4 * H), lambda i: (i, 0, 0)),
            pl.BlockSpec((1, 1, 4 * H), lambda i: (i, 0, 0)),
        ],
        out_shape=[
            jax.ShapeDtypeStruct((_NG, _PG, 4 * H), jnp.float32),
            jax.ShapeDtypeStruct((_NG, 1, 4 * H), jnp.float32),
        ],
    )(xf, xb, pf, pb, *fw_w, *bw_w)


def kernel(batch_nodes, node_features, fw_adj, bw_adj, idx_sql_seqs,
           sql_seqs_lens, embed_table,
           Wi_sf, Wh_sf, b_sf, Wi_sb, Wh_sb, b_sb, Wi_nn, Wh_nn, b_nn,
           fw0_Ws, fw0_Wn, fw0_a, fw0_Wg, fw0_Ug, fw0_b,
           bw0_Ws, bw0_Wn, bw0_a, bw0_Wg, bw0_Ug, bw0_b,
           fw1_Ws, fw1_Wn, fw1_a, fw1_Wg, fw1_Ug, fw1_b,
           bw1_Ws, bw1_Wn, bw1_a, bw1_Wg, bw1_Ug, bw1_b):
    nf_pad = jnp.pad(node_features, ((0, NPAD - N), (0, 0)))
    idx_all = jnp.concatenate([
        nf_pad.reshape(-1),
        idx_sql_seqs[:, 0],
        jnp.zeros((41216 - NPAD * 4 - 16,), jnp.int32),
    ])
    nbf = jnp.pad(fw_adj[:-1], ((0, NPAD - N), (0, 0))).reshape(-1)
    nbb = jnp.pad(bw_adj[:-1], ((0, NPAD - N), (0, 0))).reshape(-1)

    rows = _sc_gather(embed_table, idx_all, ch=184, nchunk=7)
    emb2 = rows[:NPAD * 4].reshape(NPAD, 4 * H)
    x0 = rows[NPAD * 4:NPAD * 4 + 16]

    a2 = jnp.stack([fw0_a, bw0_a], axis=1)
    h0, p02 = _tc_lstm(emb2, Wi_nn, Wh_nn, b_nn[None], a2)
    p0f = p02[:, 0]
    p0b = p02[:, 1]

    seqs = _tc_seq(x0, Wi_sf, b_sf[None], Wi_sb, b_sb[None])

    pool0f = _sc_pool(h0, nbf, p0f, c_nodes=8)
    pool0b = _sc_pool(h0, nbb, p0b, c_nodes=8)
    x1f, x1b, p1f2, p1b2 = _tc_agg0(
        h0, pool0f, pool0b,
        (fw0_Ws, fw0_Wn, fw0_Wg, fw0_Ug, fw0_b[None]),
        (bw0_Ws, bw0_Wn, bw0_Wg, bw0_Ug, bw0_b[None]),
        fw1_a[:, None], bw1_a[:, None])

    pool1f = _sc_pool(x1f, nbf, p1f2.reshape(-1), c_nodes=4)
    pool1b = _sc_pool(x1b, nbb, p1b2.reshape(-1), c_nodes=4)
    graph_hidden, graph_embedding3 = _tc_agg1(
        x1f, x1b, pool1f, pool1b,
        (fw1_Ws, fw1_Wn, fw1_Wg, fw1_Ug, fw1_b[None]),
        (bw1_Ws, bw1_Wn, bw1_Wg, bw1_Ug, bw1_b[None]))
    graph_embedding = graph_embedding3[:, 0, :]

    seqs_encoding = seqs[:, None, :]
    seqs_encoding_mask = (idx_sql_seqs == 0)
    return (graph_hidden, graph_embedding, _PG, seqs_encoding,
            seqs_encoding_mask)
